# baseline (speedup 1.0000x reference)
import functools

import jax
import jax.numpy as jnp
from jax import lax
from jax.experimental import pallas as pl
from jax.experimental.pallas import tpu as pltpu
from jax.experimental.pallas import tpu_sc as plsc


_NC = 2
_NS = 16
_NW = _NC * _NS
_LANES = 16

_CB = 8
_SPLITS = (104, 96)


def _make_sc_flatten(V, T):
    assert T == 32
    chunk_tok = 128
    n_full = V // chunk_tok
    tail = V - n_full * chunk_tok
    base_cnt = n_full // _NW
    rem = n_full % _NW
    max_cnt = base_cnt + (1 if rem else 0)
    n_pairs = (max_cnt + 1) // 2
    row_w = T * chunk_tok
    mesh = plsc.VectorSubcoreMesh(core_axis_name="c", subcore_axis_name="s")

    @functools.partial(
        pl.kernel,
        out_type=jax.ShapeDtypeStruct((V * T,), jnp.float32),
        mesh=mesh,
        compiler_params=pltpu.CompilerParams(
            use_tc_tiling_on_sc=True, needs_layout_passes=False),
        scratch_types=[
            pltpu.VMEM((T, chunk_tok), jnp.float32),
            pltpu.VMEM((T, chunk_tok), jnp.float32),
            pltpu.VMEM((row_w,), jnp.float32),
            pltpu.VMEM((row_w,), jnp.float32),
            pltpu.VMEM((max(tail, 1), chunk_tok), jnp.float32),
            pltpu.SemaphoreType.DMA,
            pltpu.SemaphoreType.DMA,
            pltpu.SemaphoreType.DMA,
            pltpu.SemaphoreType.DMA,
        ],
    )
    def sc_flatten(wt_hbm, wtail_hbm, out_hbm, st0, st1, ch0, ch1, st_tail,
                   isem0, isem1, osem0, osem1):
        wid = lax.axis_index("s") * _NC + lax.axis_index("c")
        start = wid * base_cnt + jnp.minimum(wid, rem)
        count = base_cnt + jnp.where(wid < rem, 1, 0)

        def fire_in(g, st, isem):
            pltpu.async_copy(wt_hbm.at[:, pl.ds(g * chunk_tok, chunk_tok)],
                             st, isem)

        def wait_in(st, isem):
            pltpu.make_async_copy(
                wt_hbm.at[:, pl.ds(0, chunk_tok)], st, isem).wait()

        def wait_out(ch, osem):
            pltpu.make_async_copy(
                ch, out_hbm.at[pl.ds(0, row_w)], osem).wait()

        def permute(st, ch, ntok):
            i0 = lax.iota(jnp.int32, 16)
            i1 = i0 + 16
            for c in range(ntok):
                cv = jnp.full((16,), c, jnp.int32)
                ch[pl.ds(c * T, 16)] = plsc.load_gather(st, [i0, cv])
                ch[pl.ds(c * T + 16, 16)] = plsc.load_gather(st, [i1, cv])

        def half_step(k2, st, ch, isem, osem, st_n, isem_n):
            @pl.when(k2 < count)
            def _():
                wait_in(st, isem)

                @pl.when(k2 + 1 < count)
                def _():
                    fire_in(start + k2 + 1, st_n, isem_n)

                @pl.when(k2 >= 2)
                def _():
                    wait_out(ch, osem)

                permute(st, ch, chunk_tok)
                pltpu.async_copy(
                    ch, out_hbm.at[pl.ds((start + k2) * row_w, row_w)], osem)

        fire_in(start, st0, isem0)

        def body(t, carry):
            half_step(2 * t, st0, ch0, isem0, osem0, st1, isem1)
            half_step(2 * t + 1, st1, ch1, isem1, osem1, st0, isem0)
            return carry

        lax.fori_loop(0, n_pairs, body, 0)
        wait_out(ch0, osem0)
        wait_out(ch1, osem1)

        if tail:
            @pl.when(wid == _NW - 1)
            def _():
                pltpu.sync_copy(wtail_hbm, st_tail)
                i0 = lax.iota(jnp.int32, 16)
                i1 = i0 + 16
                for c in range(tail):
                    cv = jnp.full((16,), c, jnp.int32)
                    ch0[pl.ds(c * T, 16)] = plsc.load_gather(
                        st_tail, [cv, i0])
                    ch0[pl.ds(c * T + 16, 16)] = plsc.load_gather(
                        st_tail, [cv, i1])
                pltpu.sync_copy(
                    ch0.at[pl.ds(0, tail * T)],
                    out_hbm.at[pl.ds(n_full * row_w, tail * T)])

    return sc_flatten


def _sum_block(rows_ref, out_ref, out_row0, n_rows, l_per_row):
    unroll = 4
    steps = l_per_row // unroll

    col_e = lax.iota(jnp.int32, 16) * 2
    col_o = col_e + 1

    for i in range(n_rows):
        flat0 = i * l_per_row

        def body(t, accs):
            accs = list(accs)
            r = flat0 + t * unroll
            for u in range(unroll):
                v = rows_ref[r + u, 0:32]
                a, b = plsc.unpack(v, format=plsc.PackFormat.INTERLEAVED,
                                   preferred_element_type=jnp.float32)
                accs[2 * u] = accs[2 * u] + a
                accs[2 * u + 1] = accs[2 * u + 1] + b
            return tuple(accs)

        z = jnp.zeros((_LANES,), jnp.float32)
        accs = lax.fori_loop(0, steps, body, (z,) * (2 * unroll))
        acc_e = (accs[0] + accs[2]) + (accs[4] + accs[6])
        acc_o = (accs[1] + accs[3]) + (accs[5] + accs[7])
        row = out_ref.at[out_row0 + i]
        plsc.store_scatter(row, [col_e], acc_e)
        plsc.store_scatter(row, [col_o], acc_o)


def _make_sc_embed_sum(B, V, T, L):
    assert T == 32 and sum(_SPLITS) == L
    b_per_w = B // _NW
    n_blocks = b_per_w // _CB
    rows_per_block = _CB * L
    mesh = plsc.VectorSubcoreMesh(core_axis_name="c", subcore_axis_name="s")

    @functools.partial(
        pl.kernel,
        out_type=jax.ShapeDtypeStruct((B, T), jnp.float32),
        mesh=mesh,
        compiler_params=pltpu.CompilerParams(
            use_tc_tiling_on_sc=False, needs_layout_passes=False),
        scratch_types=[
            pltpu.VMEM((2, _CB, L), jnp.int32),
            pltpu.VMEM((rows_per_block, T), jnp.bfloat16),
            pltpu.VMEM((rows_per_block, T), jnp.bfloat16),
            pltpu.VMEM((b_per_w, T), jnp.float32),
            pltpu.SemaphoreType.DMA,
            pltpu.SemaphoreType.DMA,
            pltpu.SemaphoreType.DMA,
            pltpu.SemaphoreType.DMA,
        ],
    )
    def sc_embed_sum(x_hbm, tab_hbm, out_hbm, idx_v, rows0, rows1, out_v,
                     gsem0, gsem1, isem0, isem1):
        wid = lax.axis_index("s") * _NC + lax.axis_index("c")
        base = wid * b_per_w
        rows_bufs = (rows0, rows1)
        gsems = (gsem0, gsem1)
        isems = (isem0, isem1)

        def idx_src(kb):
            return x_hbm.at[pl.ds(base + kb * _CB, _CB)]

        def fire_gathers(kb_buf, rows_ref, sem):
            for i in range(_CB):
                off = 0
                for g in _SPLITS:
                    pltpu.async_copy(
                        tab_hbm.at[idx_v.at[kb_buf, i, pl.ds(off, g)]],
                        rows_ref.at[pl.ds(i * L + off, g)],
                        sem,
                    )
                    off += g

        def drain_gathers(rows_ref, sem):
            pltpu.make_async_copy(
                tab_hbm.at[pl.ds(0, rows_per_block)], rows_ref, sem
            ).wait()

        pltpu.sync_copy(idx_src(0), idx_v.at[0])
        fire_gathers(0, rows0, gsem0)
        pltpu.async_copy(idx_src(1), idx_v.at[1], isem1)

        def half_step(kb, cur):
            rows_c = rows_bufs[cur]
            rows_n = rows_bufs[1 - cur]
            drain_gathers(rows_c, gsems[cur])

            @pl.when(kb + 2 < n_blocks)
            def _():
                pltpu.async_copy(idx_src(kb + 2), idx_v.at[cur], isems[cur])

            @pl.when(kb + 1 < n_blocks)
            def _():
                pltpu.make_async_copy(
                    idx_src(kb + 1), idx_v.at[1 - cur], isems[1 - cur]
                ).wait()
                fire_gathers(1 - cur, rows_n, gsems[1 - cur])

            _sum_block(rows_c, out_v, kb * _CB, _CB, L)

        def body(t, carry):
            half_step(2 * t, 0)
            half_step(2 * t + 1, 1)
            return carry

        lax.fori_loop(0, n_blocks // 2, body, 0)
        pltpu.sync_copy(out_v, out_hbm.at[pl.ds(base, b_per_w)])

    return sc_embed_sum


def _cast_body(i_ref, o_ref):
    o_ref[...] = i_ref[...].astype(jnp.bfloat16)


def _tc_cast_bf16(w_t):
    T, V = w_t.shape
    c = 32768
    return pl.pallas_call(
        _cast_body,
        out_shape=jax.ShapeDtypeStruct((T, V), jnp.bfloat16),
        grid=(pl.cdiv(V, c),),
        in_specs=[pl.BlockSpec((T, c), lambda i: (0, i))],
        out_specs=pl.BlockSpec((T, c), lambda i: (0, i)),
    )(w_t)


def _logsoftmax_body(s_ref, b_ref, o_ref):
    s = s_ref[...] + b_ref[...]
    m = jnp.max(s, axis=-1, keepdims=True)
    e = jnp.exp(s - m)
    lse = jnp.log(jnp.sum(e, axis=-1, keepdims=True))
    o_ref[...] = (s - m) - lse


def _tc_log_softmax(scores, bias):
    B, T = scores.shape
    blk = min(2048, B)
    return pl.pallas_call(
        _logsoftmax_body,
        out_shape=jax.ShapeDtypeStruct((B, T), jnp.float32),
        grid=(B // blk,),
        in_specs=[
            pl.BlockSpec((blk, T), lambda i: (i, 0)),
            pl.BlockSpec((1, T), lambda i: (0, 0)),
        ],
        out_specs=pl.BlockSpec((blk, T), lambda i: (i, 0)),
    )(scores, bias.reshape(1, T))


def kernel(x, embed_weight, bow_bias):
    B, L = x.shape
    V, T = embed_weight.shape
    w_lin = _tc_cast_bf16(embed_weight.T).T
    scores = _make_sc_embed_sum(B, V, T, L)(x, w_lin)
    return _tc_log_softmax(scores, bow_bias)


# speedup vs baseline: 1.2297x; 1.2297x over previous
import functools

import jax
import jax.numpy as jnp
from jax import lax
from jax.experimental import pallas as pl
from jax.experimental.pallas import tpu as pltpu
from jax.experimental.pallas import tpu_sc as plsc


_NC = 2
_NS = 16
_NW = _NC * _NS
_LANES = 16

_CB = 8
_SPLITS = (104, 96)


def _make_sc_flatten(V, T):
    assert T == 32
    chunk_tok = 128
    n_full = V // chunk_tok
    tail = V - n_full * chunk_tok
    base_cnt = n_full // _NW
    rem = n_full % _NW
    max_cnt = base_cnt + (1 if rem else 0)
    n_pairs = (max_cnt + 1) // 2
    row_w = T * chunk_tok
    mesh = plsc.VectorSubcoreMesh(core_axis_name="c", subcore_axis_name="s")

    @functools.partial(
        pl.kernel,
        out_type=jax.ShapeDtypeStruct((V * T,), jnp.float32),
        mesh=mesh,
        compiler_params=pltpu.CompilerParams(
            use_tc_tiling_on_sc=True, needs_layout_passes=False),
        scratch_types=[
            pltpu.VMEM((T, chunk_tok), jnp.float32),
            pltpu.VMEM((T, chunk_tok), jnp.float32),
            pltpu.VMEM((row_w,), jnp.float32),
            pltpu.VMEM((row_w,), jnp.float32),
            pltpu.VMEM((max(tail, 1), chunk_tok), jnp.float32),
            pltpu.SemaphoreType.DMA,
            pltpu.SemaphoreType.DMA,
            pltpu.SemaphoreType.DMA,
            pltpu.SemaphoreType.DMA,
        ],
    )
    def sc_flatten(wt_hbm, wtail_hbm, out_hbm, st0, st1, ch0, ch1, st_tail,
                   isem0, isem1, osem0, osem1):
        wid = lax.axis_index("s") * _NC + lax.axis_index("c")
        start = wid * base_cnt + jnp.minimum(wid, rem)
        count = base_cnt + jnp.where(wid < rem, 1, 0)

        def fire_in(g, st, isem):
            pltpu.async_copy(wt_hbm.at[:, pl.ds(g * chunk_tok, chunk_tok)],
                             st, isem)

        def wait_in(st, isem):
            pltpu.make_async_copy(
                wt_hbm.at[:, pl.ds(0, chunk_tok)], st, isem).wait()

        def wait_out(ch, osem):
            pltpu.make_async_copy(
                ch, out_hbm.at[pl.ds(0, row_w)], osem).wait()

        def permute(st, ch, ntok):
            i0 = lax.iota(jnp.int32, 16)
            i1 = i0 + 16
            for c in range(ntok):
                cv = jnp.full((16,), c, jnp.int32)
                ch[pl.ds(c * T, 16)] = plsc.load_gather(st, [i0, cv])
                ch[pl.ds(c * T + 16, 16)] = plsc.load_gather(st, [i1, cv])

        def half_step(k2, st, ch, isem, osem, st_n, isem_n):
            @pl.when(k2 < count)
            def _():
                wait_in(st, isem)

                @pl.when(k2 + 1 < count)
                def _():
                    fire_in(start + k2 + 1, st_n, isem_n)

                @pl.when(k2 >= 2)
                def _():
                    wait_out(ch, osem)

                permute(st, ch, chunk_tok)
                pltpu.async_copy(
                    ch, out_hbm.at[pl.ds((start + k2) * row_w, row_w)], osem)

        fire_in(start, st0, isem0)

        def body(t, carry):
            half_step(2 * t, st0, ch0, isem0, osem0, st1, isem1)
            half_step(2 * t + 1, st1, ch1, isem1, osem1, st0, isem0)
            return carry

        lax.fori_loop(0, n_pairs, body, 0)
        wait_out(ch0, osem0)
        wait_out(ch1, osem1)

        if tail:
            @pl.when(wid == _NW - 1)
            def _():
                pltpu.sync_copy(wtail_hbm, st_tail)
                i0 = lax.iota(jnp.int32, 16)
                i1 = i0 + 16
                for c in range(tail):
                    cv = jnp.full((16,), c, jnp.int32)
                    ch0[pl.ds(c * T, 16)] = plsc.load_gather(
                        st_tail, [cv, i0])
                    ch0[pl.ds(c * T + 16, 16)] = plsc.load_gather(
                        st_tail, [cv, i1])
                pltpu.sync_copy(
                    ch0.at[pl.ds(0, tail * T)],
                    out_hbm.at[pl.ds(n_full * row_w, tail * T)])

    return sc_flatten


def _sum_block(rows_ref, out_ref, out_row0, n_rows, l_per_row):
    unroll = 8
    steps = l_per_row // unroll

    for i in range(n_rows):
        flat0 = i * l_per_row

        def body(t, accs):
            accs = list(accs)
            r = flat0 + t * unroll
            for u in range(unroll):
                accs[2 * (u % 4)] = (
                    accs[2 * (u % 4)] + rows_ref[r + u, 0:16])
                accs[2 * (u % 4) + 1] = (
                    accs[2 * (u % 4) + 1] + rows_ref[r + u, 16:32])
            return tuple(accs)

        z = jnp.zeros((_LANES,), jnp.float32)
        accs = lax.fori_loop(0, steps, body, (z,) * 8)
        out_ref[out_row0 + i, 0:16] = (accs[0] + accs[2]) + (accs[4] + accs[6])
        out_ref[out_row0 + i, 16:32] = (accs[1] + accs[3]) + (accs[5] + accs[7])


def _make_sc_embed_sum(B, V, T, L):
    assert T == 32 and sum(_SPLITS) == L
    b_per_w = B // _NW
    n_blocks = b_per_w // _CB
    rows_per_block = _CB * L
    mesh = plsc.VectorSubcoreMesh(core_axis_name="c", subcore_axis_name="s")

    @functools.partial(
        pl.kernel,
        out_type=jax.ShapeDtypeStruct((B, T), jnp.float32),
        mesh=mesh,
        compiler_params=pltpu.CompilerParams(
            use_tc_tiling_on_sc=False, needs_layout_passes=False),
        scratch_types=[
            pltpu.VMEM((2, _CB, L), jnp.int32),
            pltpu.VMEM((rows_per_block, T), jnp.float32),
            pltpu.VMEM((rows_per_block, T), jnp.float32),
            pltpu.VMEM((b_per_w, T), jnp.float32),
            pltpu.SemaphoreType.DMA,
            pltpu.SemaphoreType.DMA,
            pltpu.SemaphoreType.DMA,
            pltpu.SemaphoreType.DMA,
        ],
    )
    def sc_embed_sum(x_hbm, tab_hbm, out_hbm, idx_v, rows0, rows1, out_v,
                     gsem0, gsem1, isem0, isem1):
        wid = lax.axis_index("s") * _NC + lax.axis_index("c")
        base = wid * b_per_w
        rows_bufs = (rows0, rows1)
        gsems = (gsem0, gsem1)
        isems = (isem0, isem1)

        def idx_src(kb):
            return x_hbm.at[pl.ds(base + kb * _CB, _CB)]

        def fire_gathers(kb_buf, rows_ref, sem):
            for i in range(_CB):
                off = 0
                for g in _SPLITS:
                    pltpu.async_copy(
                        tab_hbm.at[idx_v.at[kb_buf, i, pl.ds(off, g)]],
                        rows_ref.at[pl.ds(i * L + off, g)],
                        sem,
                    )
                    off += g

        def drain_gathers(rows_ref, sem):
            pltpu.make_async_copy(
                tab_hbm.at[pl.ds(0, rows_per_block)], rows_ref, sem
            ).wait()

        pltpu.sync_copy(idx_src(0), idx_v.at[0])
        fire_gathers(0, rows0, gsem0)
        pltpu.async_copy(idx_src(1), idx_v.at[1], isem1)

        def half_step(kb, cur):
            rows_c = rows_bufs[cur]
            rows_n = rows_bufs[1 - cur]
            drain_gathers(rows_c, gsems[cur])

            @pl.when(kb + 2 < n_blocks)
            def _():
                pltpu.async_copy(idx_src(kb + 2), idx_v.at[cur], isems[cur])

            @pl.when(kb + 1 < n_blocks)
            def _():
                pltpu.make_async_copy(
                    idx_src(kb + 1), idx_v.at[1 - cur], isems[1 - cur]
                ).wait()
                fire_gathers(1 - cur, rows_n, gsems[1 - cur])

            _sum_block(rows_c, out_v, kb * _CB, _CB, L)

        def body(t, carry):
            half_step(2 * t, 0)
            half_step(2 * t + 1, 1)
            return carry

        lax.fori_loop(0, n_blocks // 2, body, 0)
        pltpu.sync_copy(out_v, out_hbm.at[pl.ds(base, b_per_w)])

    return sc_embed_sum


def _cast_body(i_ref, o_ref):
    o_ref[...] = i_ref[...].astype(jnp.bfloat16)


def _tc_cast_bf16(w_t):
    T, V = w_t.shape
    c = 32768
    return pl.pallas_call(
        _cast_body,
        out_shape=jax.ShapeDtypeStruct((T, V), jnp.bfloat16),
        grid=(pl.cdiv(V, c),),
        in_specs=[pl.BlockSpec((T, c), lambda i: (0, i))],
        out_specs=pl.BlockSpec((T, c), lambda i: (0, i)),
    )(w_t)


def _logsoftmax_body(s_ref, b_ref, o_ref):
    s = s_ref[...] + b_ref[...]
    m = jnp.max(s, axis=-1, keepdims=True)
    e = jnp.exp(s - m)
    lse = jnp.log(jnp.sum(e, axis=-1, keepdims=True))
    o_ref[...] = (s - m) - lse


def _tc_log_softmax(scores, bias):
    B, T = scores.shape
    blk = min(2048, B)
    return pl.pallas_call(
        _logsoftmax_body,
        out_shape=jax.ShapeDtypeStruct((B, T), jnp.float32),
        grid=(B // blk,),
        in_specs=[
            pl.BlockSpec((blk, T), lambda i: (i, 0)),
            pl.BlockSpec((1, T), lambda i: (0, 0)),
        ],
        out_specs=pl.BlockSpec((blk, T), lambda i: (i, 0)),
    )(scores, bias.reshape(1, T))


def kernel(x, embed_weight, bow_bias):
    B, L = x.shape
    V, T = embed_weight.shape
    w_lin = embed_weight
    scores = _make_sc_embed_sum(B, V, T, L)(x, w_lin)
    return _tc_log_softmax(scores, bow_bias)
